# baseline (speedup 1.0000x reference)
import functools
import math

import jax
import jax.numpy as jnp
from jax import lax
from jax.experimental import pallas as pl
from jax.experimental.pallas import tpu as pltpu
from jax.experimental.pallas import tpu_sc as plsc

_TS = 2048
_W = 5
_WW = _W * _W
_NC = 2
_NS = 16
_NW = _NC * _NS
_MPT = 64
_GM = 4
_NBUF = 4


@functools.lru_cache(maxsize=None)
def _build_tc_transpose(n, c, hw, pad_block):
    nblk = n * hw // _TS
    nchunk = hw // _TS

    def body(x_ref, o_ref):
        if pad_block:
            j = pl.program_id(0)

            @pl.when(j < nblk)
            def _():
                o_ref[0] = jnp.transpose(x_ref[0], (1, 0))

            @pl.when(j >= nblk)
            def _():
                o_ref[0] = jnp.zeros((_TS, c), jnp.float32)
        else:
            o_ref[0] = jnp.transpose(x_ref[0], (1, 0))

    nsteps = nblk + 1 if pad_block else nblk

    def in_map(j):
        jj = jnp.minimum(j, nblk - 1)
        return (jj // nchunk, 0, jj % nchunk)

    return pl.pallas_call(
        body,
        grid=(nsteps,),
        in_specs=[pl.BlockSpec((1, c, _TS), in_map)],
        out_specs=pl.BlockSpec((1, _TS, c), lambda j: (j, 0, 0)),
        out_shape=jax.ShapeDtypeStruct((nsteps, _TS, c), jnp.float32),
    )


@functools.lru_cache(maxsize=None)
def _build_sc_gather(M, C, n, hf, wf, Lc, stride, pad):
    zrow = n * hf * wf
    full = M // _MPT
    rem = M - full * _MPT
    assert rem % _GM == 0 and full + (1 if rem else 0) <= _NW
    gsz = _GM * _WW
    mesh = plsc.VectorSubcoreMesh(core_axis_name="c", subcore_axis_name="s")

    @functools.partial(
        pl.kernel,
        mesh=mesh,
        out_type=[
            jax.ShapeDtypeStruct((M, C), jnp.float32),
            jax.ShapeDtypeStruct((M, _WW, C), jnp.float32),
        ],
        scratch_types=[
            pltpu.VMEM((_MPT,), jnp.int32),
            pltpu.VMEM((_MPT,), jnp.int32),
            pltpu.VMEM((_MPT,), jnp.int32),
            pltpu.VMEM((_MPT,), jnp.int32),
            pltpu.VMEM((_MPT // _GM, gsz), jnp.int32),
            pltpu.VMEM((_MPT, C), jnp.float32),
            pltpu.VMEM((_NBUF * gsz, C), jnp.float32),
            pltpu.SemaphoreType.DMA,
            pltpu.SemaphoreType.DMA,
            pltpu.SemaphoreType.DMA,
            pltpu.SemaphoreType.DMA,
            pltpu.SemaphoreType.DMA,
        ],
        compiler_params=pltpu.CompilerParams(needs_layout_passes=False),
    )
    def sc_gather(tab3_h, tabq_h, b_h, i_h, j_h, f3_out, fq_out,
                  bv_s, iv_s, jv_s, idx3_s, fqidx_s, f3buf_s, fqbuf_s,
                  sem3, semA, semB, semC, semD):
        wid = lax.axis_index("s") * _NC + lax.axis_index("c")
        sems = [semA, semB, semC, semD]

        def process(base, nm):
            lane = lax.iota(jnp.int32, 16)
            rowpos0 = lane >> 2
            colpos = (lane & 3) * _WW
            zrow_v = jnp.full((16,), zrow, jnp.int32)
            ncg = nm // 16
            ng = nm // _GM
            pltpu.sync_copy(b_h.at[pl.ds(base, nm)], bv_s.at[pl.ds(0, nm)])
            pltpu.sync_copy(i_h.at[pl.ds(base, nm)], iv_s.at[pl.ds(0, nm)])
            pltpu.sync_copy(j_h.at[pl.ds(base, nm)], jv_s.at[pl.ds(0, nm)])
            for cg in range(ncg):
                sl = pl.ds(cg * 16, 16)
                idx3_s[sl] = bv_s[sl] * Lc + iv_s[sl]
            h3 = pltpu.async_copy(
                tab3_h.at[idx3_s.at[pl.ds(0, nm)]],
                f3buf_s.at[pl.ds(0, nm)], sem3)
            for cg in range(ncg):
                sl = pl.ds(cg * 16, 16)
                bb = bv_s[sl] * (hf * wf)
                jv = jv_s[sl]
                grid = hf // stride
                row = jv // grid
                col = jv - row * grid
                rb = row * stride - pad
                cb = col * stride - pad
                rowpos = rowpos0 + cg * 4
                for tap in range(_WW):
                    kh, kw = divmod(tap, _W)
                    r = rb + kh
                    c = cb + kw
                    idx = bb + r * wf + c
                    idx = jnp.where((r >= 0) & (c >= 0), idx, zrow_v)
                    plsc.store_scatter(fqidx_s, [rowpos, colpos + tap], idx)
            def fire(g):
                return pltpu.async_copy(
                    tabq_h.at[fqidx_s.at[g]],
                    fqbuf_s.at[pl.ds((g % _NBUF) * gsz, gsz)], sems[g % _NBUF])

            handles = [None] * ng
            for g in range(min(_NBUF, ng)):
                handles[g] = fire(g)
            for g in range(ng):
                handles[g].wait()
                for k in range(_GM):
                    pltpu.sync_copy(
                        fqbuf_s.at[pl.ds((g % _NBUF) * gsz + k * _WW, _WW)],
                        fq_out.at[base + g * _GM + k])
                nxt = g + _NBUF
                if nxt < ng:
                    handles[nxt] = fire(nxt)
            h3.wait()
            pltpu.sync_copy(f3buf_s.at[pl.ds(0, nm)],
                            f3_out.at[pl.ds(base, nm)])

        @pl.when(wid < full)
        def _():
            process(wid * _MPT, _MPT)

        if rem:
            @pl.when(wid == full)
            def _():
                process(full * _MPT, rem)

    return sc_gather


def kernel(feat_3D, feat_query_f, b_ids, i_ids, j_ids, q_hw_f, q_hw_c):
    n, c, hf, wf = feat_query_f.shape
    Lc = feat_3D.shape[-1]
    grid = math.isqrt(Lc)
    stride = hf // grid
    pad = _W // 2
    M = b_ids.shape[0]
    hw = hf * wf
    tab3 = jnp.transpose(feat_3D, (0, 2, 1)).reshape(n * Lc, c)
    tabq = jnp.transpose(feat_query_f, (0, 2, 3, 1)).reshape(n * hw, c)
    tabq = jnp.concatenate([tabq, jnp.zeros((1, c), tabq.dtype)], axis=0)
    sc = _build_sc_gather(M, c, n, hf, wf, Lc, stride, pad)
    f3_flat, fq_3d = sc(tab3, tabq,
                          b_ids.astype(jnp.int32),
                          i_ids.astype(jnp.int32),
                          j_ids.astype(jnp.int32))
    return (f3_flat.reshape(M, c, 1), fq_3d)


# speedup vs baseline: 1.0043x; 1.0043x over previous
import functools
import math

import jax
import jax.numpy as jnp
from jax import lax
from jax.experimental import pallas as pl
from jax.experimental.pallas import tpu as pltpu
from jax.experimental.pallas import tpu_sc as plsc

_TS = 2048
_W = 5
_WW = _W * _W
_NC = 2
_NS = 16
_NW = _NC * _NS
_MPT = 64
_GM = 4
_NBUF = 6


@functools.lru_cache(maxsize=None)
def _build_tc_transpose(n, c, hw, pad_block):
    nblk = n * hw // _TS
    nchunk = hw // _TS

    def body(x_ref, o_ref):
        if pad_block:
            j = pl.program_id(0)

            @pl.when(j < nblk)
            def _():
                o_ref[0] = jnp.transpose(x_ref[0], (1, 0))

            @pl.when(j >= nblk)
            def _():
                o_ref[0] = jnp.zeros((_TS, c), jnp.float32)
        else:
            o_ref[0] = jnp.transpose(x_ref[0], (1, 0))

    nsteps = nblk + 1 if pad_block else nblk

    def in_map(j):
        jj = jnp.minimum(j, nblk - 1)
        return (jj // nchunk, 0, jj % nchunk)

    return pl.pallas_call(
        body,
        grid=(nsteps,),
        in_specs=[pl.BlockSpec((1, c, _TS), in_map)],
        out_specs=pl.BlockSpec((1, _TS, c), lambda j: (j, 0, 0)),
        out_shape=jax.ShapeDtypeStruct((nsteps, _TS, c), jnp.float32),
    )


@functools.lru_cache(maxsize=None)
def _build_sc_gather(M, C, n, hf, wf, Lc, stride, pad):
    zrow = n * hf * wf
    full = M // _MPT
    rem = M - full * _MPT
    assert rem % _GM == 0 and full + (1 if rem else 0) <= _NW
    gsz = _GM * _WW
    mesh = plsc.VectorSubcoreMesh(core_axis_name="c", subcore_axis_name="s")

    @functools.partial(
        pl.kernel,
        mesh=mesh,
        out_type=[
            jax.ShapeDtypeStruct((M, C), jnp.float32),
            jax.ShapeDtypeStruct((M, _WW, C), jnp.float32),
        ],
        scratch_types=[
            pltpu.VMEM((_MPT,), jnp.int32),
            pltpu.VMEM((_MPT,), jnp.int32),
            pltpu.VMEM((_MPT,), jnp.int32),
            pltpu.VMEM((_MPT,), jnp.int32),
            pltpu.VMEM((_MPT // _GM, gsz), jnp.int32),
            pltpu.VMEM((_MPT, C), jnp.float32),
            pltpu.VMEM((_NBUF * gsz, C), jnp.float32),
        ] + [pltpu.SemaphoreType.DMA] * (1 + 2 * _NBUF),
        compiler_params=pltpu.CompilerParams(needs_layout_passes=False),
    )
    def sc_gather(tab3_h, tabq_h, b_h, i_h, j_h, f3_out, fq_out,
                  bv_s, iv_s, jv_s, idx3_s, fqidx_s, f3buf_s, fqbuf_s,
                  sem3, *sems_all):
        wid = lax.axis_index("s") * _NC + lax.axis_index("c")
        gsems = list(sems_all[:_NBUF])
        osems = list(sems_all[_NBUF:])

        def process(base, nm):
            lane = lax.iota(jnp.int32, 16)
            rowpos0 = lane >> 2
            colpos = (lane & 3) * _WW
            zrow_v = jnp.full((16,), zrow, jnp.int32)
            ncg = nm // 16
            ng = nm // _GM
            pltpu.sync_copy(b_h.at[pl.ds(base, nm)], bv_s.at[pl.ds(0, nm)])
            pltpu.sync_copy(i_h.at[pl.ds(base, nm)], iv_s.at[pl.ds(0, nm)])
            pltpu.sync_copy(j_h.at[pl.ds(base, nm)], jv_s.at[pl.ds(0, nm)])
            for cg in range(ncg):
                sl = pl.ds(cg * 16, 16)
                idx3_s[sl] = bv_s[sl] * Lc + iv_s[sl]
            h3 = pltpu.async_copy(
                tab3_h.at[idx3_s.at[pl.ds(0, nm)]],
                f3buf_s.at[pl.ds(0, nm)], sem3)
            for cg in range(ncg):
                sl = pl.ds(cg * 16, 16)
                bb = bv_s[sl] * (hf * wf)
                jv = jv_s[sl]
                grid = hf // stride
                row = jv // grid
                col = jv - row * grid
                rb = row * stride - pad
                cb = col * stride - pad
                rowpos = rowpos0 + cg * 4
                for tap in range(_WW):
                    kh, kw = divmod(tap, _W)
                    r = rb + kh
                    c = cb + kw
                    idx = bb + r * wf + c
                    idx = jnp.where((r >= 0) & (c >= 0), idx, zrow_v)
                    plsc.store_scatter(fqidx_s, [rowpos, colpos + tap], idx)
            def fire(g):
                return pltpu.async_copy(
                    tabq_h.at[fqidx_s.at[g]],
                    fqbuf_s.at[pl.ds((g % _NBUF) * gsz, gsz)], gsems[g % _NBUF])

            def fire_outs(g):
                hs = []
                for k in range(_GM):
                    hs.append(pltpu.async_copy(
                        fqbuf_s.at[pl.ds((g % _NBUF) * gsz + k * _WW, _WW)],
                        fq_out.at[base + g * _GM + k], osems[g % _NBUF]))
                return hs

            depth = min(_NBUF - 2, ng)
            ghandles = [None] * ng
            ohandles = [None] * ng
            for g in range(depth):
                ghandles[g] = fire(g)
            for g in range(ng):
                ghandles[g].wait()
                ohandles[g] = fire_outs(g)
                if g >= 2:
                    for h in ohandles[g - 2]:
                        h.wait()
                nxt = g + depth
                if nxt < ng:
                    ghandles[nxt] = fire(nxt)
            for g in range(max(ng - 2, 0), ng):
                for h in ohandles[g]:
                    h.wait()
            h3.wait()
            pltpu.sync_copy(f3buf_s.at[pl.ds(0, nm)],
                            f3_out.at[pl.ds(base, nm)])

        @pl.when(wid < full)
        def _():
            process(wid * _MPT, _MPT)

        if rem:
            @pl.when(wid == full)
            def _():
                process(full * _MPT, rem)

    return sc_gather


def kernel(feat_3D, feat_query_f, b_ids, i_ids, j_ids, q_hw_f, q_hw_c):
    n, c, hf, wf = feat_query_f.shape
    Lc = feat_3D.shape[-1]
    grid = math.isqrt(Lc)
    stride = hf // grid
    pad = _W // 2
    M = b_ids.shape[0]
    hw = hf * wf
    tab3 = jnp.transpose(feat_3D, (0, 2, 1)).reshape(n * Lc, c)
    tabq = jnp.transpose(feat_query_f, (0, 2, 3, 1)).reshape(n * hw, c)
    tabq = jnp.concatenate([tabq, jnp.zeros((1, c), tabq.dtype)], axis=0)
    sc = _build_sc_gather(M, c, n, hf, wf, Lc, stride, pad)
    f3_flat, fq_3d = sc(tab3, tabq,
                          b_ids.astype(jnp.int32),
                          i_ids.astype(jnp.int32),
                          j_ids.astype(jnp.int32))
    return (f3_flat.reshape(M, c, 1), fq_3d)


# speedup vs baseline: 1.1864x; 1.1813x over previous
import functools
import math

import jax
import jax.numpy as jnp
from jax import lax
from jax.experimental import pallas as pl
from jax.experimental.pallas import tpu as pltpu
from jax.experimental.pallas import tpu_sc as plsc

_TS = 2048
_W = 5
_WW = _W * _W
_NC = 2
_NS = 16
_NW = _NC * _NS
_MPT = 64
_GM = 4
_NBUF = 6


@functools.lru_cache(maxsize=None)
def _build_tc_transpose(n, c, hw, pad_block):
    nblk = n * hw // _TS
    nchunk = hw // _TS

    def body(x_ref, o_ref):
        if pad_block:
            j = pl.program_id(0)

            @pl.when(j < nblk)
            def _():
                o_ref[0] = jnp.transpose(x_ref[0], (1, 0))

            @pl.when(j >= nblk)
            def _():
                o_ref[0] = jnp.zeros((_TS, c), jnp.float32)
        else:
            o_ref[0] = jnp.transpose(x_ref[0], (1, 0))

    nsteps = nblk + 1 if pad_block else nblk

    def in_map(j):
        jj = jnp.minimum(j, nblk - 1)
        return (jj // nchunk, 0, jj % nchunk)

    return pl.pallas_call(
        body,
        grid=(nsteps,),
        in_specs=[pl.BlockSpec((1, c, _TS), in_map)],
        out_specs=pl.BlockSpec((1, _TS, c), lambda j: (j, 0, 0)),
        out_shape=jax.ShapeDtypeStruct((nsteps, _TS, c), jnp.float32),
    )


@functools.lru_cache(maxsize=None)
def _build_sc_gather(M, C, n, hf, wf, Lc, stride, pad):
    zrow = n * hf * wf
    full = M // _MPT
    rem = M - full * _MPT
    assert rem % _GM == 0 and full + (1 if rem else 0) <= _NW
    gsz = _GM * _WW
    mesh = plsc.VectorSubcoreMesh(core_axis_name="c", subcore_axis_name="s")

    @functools.partial(
        pl.kernel,
        mesh=mesh,
        out_type=[
            jax.ShapeDtypeStruct((M, C), jnp.float32),
            jax.ShapeDtypeStruct((M, _WW, C), jnp.float32),
        ],
        scratch_types=[
            pltpu.VMEM((_MPT,), jnp.int32),
            pltpu.VMEM((_MPT,), jnp.int32),
            pltpu.VMEM((_MPT,), jnp.int32),
            pltpu.VMEM((_MPT,), jnp.int32),
            pltpu.VMEM((_MPT // _GM, gsz), jnp.int32),
            pltpu.VMEM((_MPT, C), jnp.float32),
            pltpu.VMEM((_NBUF * gsz, C), jnp.float32),
            pltpu.VMEM(((_MPT // 16) * _WW, 16), jnp.int32),
            pltpu.VMEM(((_MPT // 16) * _WW, 16), jnp.int32),
            pltpu.VMEM((C,), jnp.float32),
        ] + [pltpu.SemaphoreType.DMA] * (2 + 2 * _NBUF),
        compiler_params=pltpu.CompilerParams(needs_layout_passes=False),
    )
    def sc_gather(tab3_h, tabq_h, b_h, i_h, j_h, f3_out, fq_out,
                  bv_s, iv_s, jv_s, idx3_s, fqidx_s, f3buf_s, fqbuf_s,
                  oob_s, cnts_s, zbuf_s, sem3, fixsem, *sems_all):
        wid = lax.axis_index("s") * _NC + lax.axis_index("c")
        gsems = list(sems_all[:_NBUF])
        osems = list(sems_all[_NBUF:])

        def process(base, nm):
            lane = lax.iota(jnp.int32, 16)
            rowpos0 = lane >> 2
            colpos = (lane & 3) * _WW
            zero_v = jnp.zeros((16,), jnp.int32)
            total_v = jnp.zeros((16,), jnp.int32)
            ncg = nm // 16
            ng = nm // _GM
            pltpu.sync_copy(b_h.at[pl.ds(base, nm)], bv_s.at[pl.ds(0, nm)])
            pltpu.sync_copy(i_h.at[pl.ds(base, nm)], iv_s.at[pl.ds(0, nm)])
            pltpu.sync_copy(j_h.at[pl.ds(base, nm)], jv_s.at[pl.ds(0, nm)])
            for cg in range(ncg):
                sl = pl.ds(cg * 16, 16)
                idx3_s[sl] = bv_s[sl] * Lc + iv_s[sl]
            h3 = pltpu.async_copy(
                tab3_h.at[idx3_s.at[pl.ds(0, nm)]],
                f3buf_s.at[pl.ds(0, nm)], sem3)
            for cg in range(ncg):
                sl = pl.ds(cg * 16, 16)
                bb = bv_s[sl] * (hf * wf)
                jv = jv_s[sl]
                grid = hf // stride
                row = jv // grid
                col = jv - row * grid
                rb = row * stride - pad
                cb = col * stride - pad
                rowpos = rowpos0 + cg * 4
                posb = (lane + (base + cg * 16)) * 32
                for tap in range(_WW):
                    kh, kw = divmod(tap, _W)
                    r = rb + kh
                    c = cb + kw
                    idx = bb + r * wf + c
                    oobm = (r < 0) | (c < 0)
                    idx = jnp.where(oobm, zero_v, idx)
                    plsc.store_scatter(fqidx_s, [rowpos, colpos + tap], idx)
                    rr = cg * _WW + tap
                    cntv = plsc.all_reduce_population_count(oobm)
                    plsc.store_compressed(oob_s.at[rr], posb + tap, mask=oobm)
                    cnts_s[rr] = cntv
                    total_v = total_v + cntv
            def fire(g):
                return pltpu.async_copy(
                    tabq_h.at[fqidx_s.at[g]],
                    fqbuf_s.at[pl.ds((g % _NBUF) * gsz, gsz)], gsems[g % _NBUF])

            def fire_outs(g):
                hs = []
                for k in range(_GM):
                    hs.append(pltpu.async_copy(
                        fqbuf_s.at[pl.ds((g % _NBUF) * gsz + k * _WW, _WW)],
                        fq_out.at[base + g * _GM + k], osems[g % _NBUF]))
                return hs

            depth = min(_NBUF - 2, ng)
            ghandles = [None] * ng
            ohandles = [None] * ng
            for g in range(depth):
                ghandles[g] = fire(g)
            for g in range(ng):
                ghandles[g].wait()
                ohandles[g] = fire_outs(g)
                if g >= 2:
                    for h in ohandles[g - 2]:
                        h.wait()
                nxt = g + depth
                if nxt < ng:
                    ghandles[nxt] = fire(nxt)
            for g in range(max(ng - 2, 0), ng):
                for h in ohandles[g]:
                    h.wait()
            h3.wait()
            pltpu.sync_copy(f3buf_s.at[pl.ds(0, nm)],
                            f3_out.at[pl.ds(base, nm)])
            for kk in range(C // 16):
                zbuf_s[pl.ds(kk * 16, 16)] = jnp.zeros((16,), jnp.float32)

            def fix_row(rr2, carry):
                cv = cnts_s[rr2, pl.ds(0, 16)]
                pv = oob_s[rr2, pl.ds(0, 16)]
                cnt_r = cv[0]
                for lane_i in range(16):
                    @pl.when(lane_i < cnt_r)
                    def _():
                        pos = pv[lane_i]
                        pltpu.async_copy(zbuf_s,
                                         fq_out.at[pos >> 5, pos & 31],
                                         fixsem)
                return carry

            lax.fori_loop(0, ncg * _WW, fix_row, 0)

            def fix_drain(i, c2):
                pltpu.make_async_copy(tab3_h.at[0], zbuf_s, fixsem).wait()
                return c2

            lax.fori_loop(0, jnp.max(total_v), fix_drain, 0)

        @pl.when(wid < full)
        def _():
            process(wid * _MPT, _MPT)

        if rem:
            @pl.when(wid == full)
            def _():
                process(full * _MPT, rem)

    return sc_gather


def kernel(feat_3D, feat_query_f, b_ids, i_ids, j_ids, q_hw_f, q_hw_c):
    n, c, hf, wf = feat_query_f.shape
    Lc = feat_3D.shape[-1]
    grid = math.isqrt(Lc)
    stride = hf // grid
    pad = _W // 2
    M = b_ids.shape[0]
    hw = hf * wf
    tab3 = jnp.transpose(feat_3D, (0, 2, 1)).reshape(n * Lc, c)
    tabq = jnp.transpose(feat_query_f, (0, 2, 3, 1)).reshape(n * hw, c)
    sc = _build_sc_gather(M, c, n, hf, wf, Lc, stride, pad)
    f3_flat, fq_3d = sc(tab3, tabq,
                          b_ids.astype(jnp.int32),
                          i_ids.astype(jnp.int32),
                          j_ids.astype(jnp.int32))
    return (f3_flat.reshape(M, c, 1), fq_3d)
